# Initial kernel scaffold; baseline (speedup 1.0000x reference)
import jax
import jax.numpy as jnp
from jax.experimental import pallas as pl


def kernel(x, lut_weight):
    raise NotImplementedError("write your pallas kernel here")


# speedup vs baseline: 1.4687x; 1.4687x over previous
import jax
import jax.numpy as jnp
from jax.experimental import pallas as pl
from jax.experimental.pallas import tpu as pltpu

BLK_P = 256


def _add_body(x_ref, lut_ref, o_ref):
    o_ref[...] = x_ref[...] + lut_ref[...]


def kernel(x, lut_weight):
    B, P, D = x.shape
    grid = (P // BLK_P, B)
    return pl.pallas_call(
        _add_body,
        grid=grid,
        in_specs=[
            pl.BlockSpec((1, BLK_P, D), lambda i, j: (j, i, 0)),
            pl.BlockSpec((BLK_P, D), lambda i, j: (i, 0)),
        ],
        out_specs=pl.BlockSpec((1, BLK_P, D), lambda i, j: (j, i, 0)),
        out_shape=jax.ShapeDtypeStruct((B, P, D), x.dtype),
    )(x, lut_weight)


# speedup vs baseline: 1.9294x; 1.3137x over previous
import jax
import jax.numpy as jnp
from jax.experimental import pallas as pl
from jax.experimental.pallas import tpu as pltpu

BLK_P = 512


def _add_body(x_ref, lut_ref, o_ref):
    o_ref[...] = x_ref[...] + lut_ref[...]


def kernel(x, lut_weight):
    B, P, D = x.shape
    grid = (P // BLK_P, B)
    return pl.pallas_call(
        _add_body,
        grid=grid,
        in_specs=[
            pl.BlockSpec((1, BLK_P, D), lambda i, j: (j, i, 0)),
            pl.BlockSpec((BLK_P, D), lambda i, j: (i, 0)),
        ],
        out_specs=pl.BlockSpec((1, BLK_P, D), lambda i, j: (j, i, 0)),
        out_shape=jax.ShapeDtypeStruct((B, P, D), x.dtype),
    )(x, lut_weight)


# speedup vs baseline: 2.1101x; 1.0937x over previous
import jax
import jax.numpy as jnp
from jax.experimental import pallas as pl
from jax.experimental.pallas import tpu as pltpu

BLK_P = 1024


def _add_body(x_ref, lut_ref, o_ref):
    o_ref[...] = x_ref[...] + lut_ref[...]


def kernel(x, lut_weight):
    B, P, D = x.shape
    grid = (P // BLK_P, B)
    return pl.pallas_call(
        _add_body,
        grid=grid,
        in_specs=[
            pl.BlockSpec((1, BLK_P, D), lambda i, j: (j, i, 0)),
            pl.BlockSpec((BLK_P, D), lambda i, j: (i, 0)),
        ],
        out_specs=pl.BlockSpec((1, BLK_P, D), lambda i, j: (j, i, 0)),
        out_shape=jax.ShapeDtypeStruct((B, P, D), x.dtype),
    )(x, lut_weight)


# speedup vs baseline: 2.2860x; 1.0834x over previous
import jax
import jax.numpy as jnp
from jax.experimental import pallas as pl
from jax.experimental.pallas import tpu as pltpu

BLK_P = 2048


def _add_body(x_ref, lut_ref, o_ref):
    o_ref[...] = x_ref[...] + lut_ref[...]


def kernel(x, lut_weight):
    B, P, D = x.shape
    grid = (P // BLK_P, B)
    return pl.pallas_call(
        _add_body,
        grid=grid,
        in_specs=[
            pl.BlockSpec((1, BLK_P, D), lambda i, j: (j, i, 0)),
            pl.BlockSpec((BLK_P, D), lambda i, j: (i, 0)),
        ],
        out_specs=pl.BlockSpec((1, BLK_P, D), lambda i, j: (j, i, 0)),
        out_shape=jax.ShapeDtypeStruct((B, P, D), x.dtype),
    )(x, lut_weight)
